# baseline (speedup 1.0000x reference)
import functools

import jax
import jax.numpy as jnp
from jax import lax
from jax.experimental import pallas as pl
from jax.experimental.pallas import tpu as pltpu
from jax.experimental.pallas import tpu_sc as plsc

_NUM_ACTION = 125
_NUM_NOUN = 352
_DBINS = 5
_BS = 64
_C = 64 + _NUM_ACTION + _NUM_NOUN
_NCH = _C * _DBINS
_SPATIAL = 256
_CONF0 = 63 * _DBINS
_CLS0 = 64 * _DBINS
_NCLS = _NUM_ACTION + _NUM_NOUN
_NPAD = 480
_NW = 32
_BPW = _BS // _NW


_NCELL = _NCH - 256


def _sc_gather_body(pred_hbm, conf_hbm, out_hbm, conf_v, cell0_v, cell1_v,
                    rows_v, red_f, red_i, sem0, sem1):
    wid = lax.axis_index("c") * 16 + lax.axis_index("s")
    lanes = lax.iota(jnp.int32, 16)
    b0 = wid * _BPW
    pltpu.sync_copy(
        conf_hbm.at[pl.ds(b0 * 1280, _BPW * 1280)], conf_v)

    def amax(i):
        def amax_body(j, carry):
            bv, bf = carry
            v = conf_v[pl.ds(i * 1280 + j * 16, 16)]
            sig = 1.0 / (1.0 + jnp.exp(-v))
            f = 16 * j + lanes
            better = (sig > bv) | ((sig == bv) & (f < bf))
            return jnp.where(better, sig, bv), jnp.where(better, f, bf)

        bv, bf = lax.fori_loop(
            0, 1280 // 16, amax_body,
            (jnp.full((16,), -1.0, jnp.float32), jnp.zeros((16,), jnp.int32)))

        for k in (1, 2, 4, 8):
            red_f[...] = bv
            red_i[...] = bf
            perm = lanes ^ k
            ov = plsc.load_gather(red_f, [perm])
            of = plsc.load_gather(red_i, [perm])
            better = (ov > bv) | ((ov == bv) & (of < bf))
            bv = jnp.where(better, ov, bv)
            bf = jnp.where(better, of, bf)
        fm = bf[0]
        return fm % _DBINS, fm // _DBINS

    d0, s0 = amax(0)
    cp0 = pltpu.make_async_copy(
        pred_hbm.at[pl.ds(b0 * _SPATIAL + (s0 // 8) * 8, 8),
                    pl.ds(256, _NCELL)], cell0_v, sem0)
    cp0.start()
    d1, s1 = amax(1)
    cp1 = pltpu.make_async_copy(
        pred_hbm.at[pl.ds((b0 + 1) * _SPATIAL + (s1 // 8) * 8, 8),
                    pl.ds(256, _NCELL)], cell1_v, sem1)
    cp1.start()

    for i, (cp, cell_v, d, s) in enumerate(
            ((cp0, cell0_v, d0, s0), (cp1, cell1_v, d1, s1))):
        cp.wait()
        srow = jnp.broadcast_to(s % 8, (16,))

        def sel_body(k, carry):
            idxc = (_CLS0 - 256) + d + 5 * (16 * k + lanes)
            idxc = jnp.minimum(idxc, _NCELL - 1)
            rows_v[pl.ds(k * 16, 16)] = plsc.load_gather(cell_v, [srow, idxc])
            return carry

        lax.fori_loop(0, _NPAD // 16, sel_body, 0)
        pltpu.sync_copy(rows_v, out_hbm.at[b0 + i])


@functools.cache
def _sc_gather():
    return pl.kernel(
        _sc_gather_body,
        out_type=jax.ShapeDtypeStruct((_BS, _NPAD), jnp.float32),
        mesh=plsc.VectorSubcoreMesh(
            core_axis_name="c", subcore_axis_name="s",
            num_cores=2, num_subcores=16),
        compiler_params=pltpu.CompilerParams(
            needs_layout_passes=False, use_tc_tiling_on_sc=True),
        scratch_types=[
            pltpu.VMEM((_BPW * 1280,), jnp.float32),
            pltpu.VMEM((8, _NCELL), jnp.float32),
            pltpu.VMEM((8, _NCELL), jnp.float32),
            pltpu.VMEM((_NPAD,), jnp.float32),
            pltpu.VMEM((16,), jnp.float32),
            pltpu.VMEM((16,), jnp.int32),
            pltpu.SemaphoreType.DMA,
            pltpu.SemaphoreType.DMA,
        ],
    )


def _ce_body(chosen_ref, ga_ref, gn_ref, out_ref):
    x = chosen_ref[...]
    lane = lax.broadcasted_iota(jnp.int32, (_BS, _NPAD), 1)
    mask_a = lane < _NUM_ACTION
    mask_n = (lane >= _NUM_ACTION) & (lane < _NCLS)
    neg = jnp.float32(-1e30)
    xa = jnp.where(mask_a, x, neg)
    xn = jnp.where(mask_n, x, neg)
    ma = jnp.max(xa, axis=1, keepdims=True)
    mn = jnp.max(xn, axis=1, keepdims=True)
    sa = jnp.sum(jnp.where(mask_a, jnp.exp(xa - ma), 0.0), axis=1, keepdims=True)
    sn = jnp.sum(jnp.where(mask_n, jnp.exp(xn - mn), 0.0), axis=1, keepdims=True)
    lse_a = ma + jnp.log(sa)
    lse_n = mn + jnp.log(sn)
    pa = jnp.sum(jnp.where(lane == ga_ref[...], x, 0.0), axis=1, keepdims=True)
    pn = jnp.sum(jnp.where(lane == gn_ref[...] + _NUM_ACTION, x, 0.0),
                 axis=1, keepdims=True)
    la = jnp.sum(lse_a - pa) * 0.5
    ln = jnp.sum(lse_n - pn) * 0.5
    out_ref[0] = la + ln
    out_ref[1] = la
    out_ref[2] = ln


_ce_call = pl.pallas_call(
    _ce_body,
    out_shape=jax.ShapeDtypeStruct((3,), jnp.float32),
    out_specs=pl.BlockSpec(memory_space=pltpu.MemorySpace.SMEM),
)


def kernel(pred, action_gt, noun_gt):
    pred_t = jnp.transpose(pred, (0, 2, 3, 1))
    pred_m = pred_t.reshape(_BS * _SPATIAL, _NCH)
    conf = pred_t[:, :, :, _CONF0:_CONF0 + _DBINS].reshape(-1)
    chosen = _sc_gather()(pred_m, conf)
    ga = action_gt.astype(jnp.int32).reshape(_BS, 1)
    gn = noun_gt.astype(jnp.int32).reshape(_BS, 1)
    out = _ce_call(chosen, ga, gn)
    return (out[0], out[1], out[2])


# speedup vs baseline: 1.3378x; 1.3378x over previous
import functools

import jax
import jax.numpy as jnp
from jax import lax
from jax.experimental import pallas as pl
from jax.experimental.pallas import tpu as pltpu
from jax.experimental.pallas import tpu_sc as plsc

_NUM_ACTION = 125
_NUM_NOUN = 352
_DBINS = 5
_BS = 64
_C = 64 + _NUM_ACTION + _NUM_NOUN
_NCH = _C * _DBINS
_SPATIAL = 256
_CONF0 = 63 * _DBINS
_CLS0 = 64 * _DBINS
_NCLS = _NUM_ACTION + _NUM_NOUN
_NPAD = 480
_NW = 32
_BPW = _BS // _NW


_NCELL = _NCH - 256


def _sc_gather_body(pred_hbm, out_hbm, conf_v, cell0_v, cell1_v,
                    rows_v, red_f, red_i, semc, semg0, semg1):
    wid = lax.axis_index("c") * 16 + lax.axis_index("s")
    lanes = lax.iota(jnp.int32, 16)
    b0 = wid * _BPW
    row0 = b0 * _SPATIAL
    row1 = (b0 + 1) * _SPATIAL
    cc = pltpu.make_async_copy(
        pred_hbm.at[pl.ds(row0, 2 * _SPATIAL), pl.ds(256, 128)], conf_v, semc)
    cc.start()

    def amax(i):
        base = i * _SPATIAL

        def amax_body(j, carry):
            bv, bf = carry
            s_vec = 16 * (j % 16) + lanes
            col = jnp.broadcast_to(59 + j // 16, (16,))
            v = plsc.load_gather(conf_v, [base + s_vec, col])
            sig = 1.0 / (1.0 + jnp.exp(-v))
            f = 5 * s_vec + (j // 16)
            better = (sig > bv) | ((sig == bv) & (f < bf))
            return jnp.where(better, sig, bv), jnp.where(better, f, bf)

        bv, bf = lax.fori_loop(
            0, (_DBINS * _SPATIAL) // 16, amax_body,
            (jnp.full((16,), -1.0, jnp.float32), jnp.zeros((16,), jnp.int32)))

        for k in (1, 2, 4, 8):
            red_f[...] = bv
            red_i[...] = bf
            perm = lanes ^ k
            ov = plsc.load_gather(red_f, [perm])
            of = plsc.load_gather(red_i, [perm])
            better = (ov > bv) | ((ov == bv) & (of < bf))
            bv = jnp.where(better, ov, bv)
            bf = jnp.where(better, of, bf)
        fm = bf[0]
        return fm % _DBINS, fm // _DBINS

    cc.wait()
    d0, s0 = amax(0)
    cp0 = pltpu.make_async_copy(
        pred_hbm.at[pl.ds(row0 + (s0 // 8) * 8, 8), pl.ds(256, _NCELL)],
        cell0_v, semg0)
    cp0.start()
    d1, s1 = amax(1)
    cp1 = pltpu.make_async_copy(
        pred_hbm.at[pl.ds(row1 + (s1 // 8) * 8, 8), pl.ds(256, _NCELL)],
        cell1_v, semg1)
    cp1.start()

    for i, (cp, cell_v, d, s) in enumerate(
            ((cp0, cell0_v, d0, s0), (cp1, cell1_v, d1, s1))):
        cp.wait()
        srow = jnp.broadcast_to(s % 8, (16,))

        def sel_body(k, carry):
            idxc = (_CLS0 - 256) + d + 5 * (16 * k + lanes)
            idxc = jnp.minimum(idxc, _NCELL - 1)
            rows_v[pl.ds(k * 16, 16)] = plsc.load_gather(cell_v, [srow, idxc])
            return carry

        lax.fori_loop(0, _NPAD // 16, sel_body, 0)
        pltpu.sync_copy(rows_v, out_hbm.at[b0 + i])


@functools.cache
def _sc_gather():
    return pl.kernel(
        _sc_gather_body,
        out_type=jax.ShapeDtypeStruct((_BS, _NPAD), jnp.float32),
        mesh=plsc.VectorSubcoreMesh(
            core_axis_name="c", subcore_axis_name="s",
            num_cores=2, num_subcores=16),
        compiler_params=pltpu.CompilerParams(
            needs_layout_passes=False, use_tc_tiling_on_sc=True),
        scratch_types=[
            pltpu.VMEM((2 * _SPATIAL, 128), jnp.float32),
            pltpu.VMEM((8, _NCELL), jnp.float32),
            pltpu.VMEM((8, _NCELL), jnp.float32),
            pltpu.VMEM((_NPAD,), jnp.float32),
            pltpu.VMEM((16,), jnp.float32),
            pltpu.VMEM((16,), jnp.int32),
            pltpu.SemaphoreType.DMA,
            pltpu.SemaphoreType.DMA,
            pltpu.SemaphoreType.DMA,
        ],
    )


def _ce_body(chosen_ref, ga_ref, gn_ref, out_ref):
    x = chosen_ref[...]
    lane = lax.broadcasted_iota(jnp.int32, (_BS, _NPAD), 1)
    mask_a = lane < _NUM_ACTION
    mask_n = (lane >= _NUM_ACTION) & (lane < _NCLS)
    neg = jnp.float32(-1e30)
    xa = jnp.where(mask_a, x, neg)
    xn = jnp.where(mask_n, x, neg)
    ma = jnp.max(xa, axis=1, keepdims=True)
    mn = jnp.max(xn, axis=1, keepdims=True)
    sa = jnp.sum(jnp.where(mask_a, jnp.exp(xa - ma), 0.0), axis=1, keepdims=True)
    sn = jnp.sum(jnp.where(mask_n, jnp.exp(xn - mn), 0.0), axis=1, keepdims=True)
    lse_a = ma + jnp.log(sa)
    lse_n = mn + jnp.log(sn)
    pa = jnp.sum(jnp.where(lane == ga_ref[...], x, 0.0), axis=1, keepdims=True)
    pn = jnp.sum(jnp.where(lane == gn_ref[...] + _NUM_ACTION, x, 0.0),
                 axis=1, keepdims=True)
    la = jnp.sum(lse_a - pa) * 0.5
    ln = jnp.sum(lse_n - pn) * 0.5
    out_ref[0] = la + ln
    out_ref[1] = la
    out_ref[2] = ln


_ce_call = pl.pallas_call(
    _ce_body,
    out_shape=jax.ShapeDtypeStruct((3,), jnp.float32),
    out_specs=pl.BlockSpec(memory_space=pltpu.MemorySpace.SMEM),
)


def kernel(pred, action_gt, noun_gt):
    pred_m = jnp.transpose(pred, (0, 2, 3, 1)).reshape(_BS * _SPATIAL, _NCH)
    chosen = _sc_gather()(pred_m)
    ga = action_gt.astype(jnp.int32).reshape(_BS, 1)
    gn = noun_gt.astype(jnp.int32).reshape(_BS, 1)
    out = _ce_call(chosen, ga, gn)
    return (out[0], out[1], out[2])


# speedup vs baseline: 1.3577x; 1.0149x over previous
import functools

import jax
import jax.numpy as jnp
from jax import lax
from jax.experimental import pallas as pl
from jax.experimental.pallas import tpu as pltpu
from jax.experimental.pallas import tpu_sc as plsc

_NUM_ACTION = 125
_NUM_NOUN = 352
_DBINS = 5
_BS = 64
_C = 64 + _NUM_ACTION + _NUM_NOUN
_NCH = _C * _DBINS
_SPATIAL = 256
_CONF0 = 63 * _DBINS
_CLS0 = 64 * _DBINS
_NCLS = _NUM_ACTION + _NUM_NOUN
_NPAD = 480
_NW = 32
_BPW = _BS // _NW


_NCELL = _NCH - 256


def _sc_gather_body(pred_hbm, out_hbm, conf_v, cell0_v, cell1_v,
                    rows_v, red_f, red_i, semc, semc1, semg0, semg1):
    wid = lax.axis_index("c") * 16 + lax.axis_index("s")
    lanes = lax.iota(jnp.int32, 16)
    b0 = wid * _BPW
    row0 = b0 * _SPATIAL
    row1 = (b0 + 1) * _SPATIAL
    cc0 = pltpu.make_async_copy(
        pred_hbm.at[pl.ds(row0, _SPATIAL), pl.ds(256, 128)],
        conf_v.at[pl.ds(0, _SPATIAL), :], semc)
    cc1 = pltpu.make_async_copy(
        pred_hbm.at[pl.ds(row1, _SPATIAL), pl.ds(256, 128)],
        conf_v.at[pl.ds(_SPATIAL, _SPATIAL), :], semc1)
    cc0.start()
    cc1.start()

    def amax(i):
        base = i * _SPATIAL

        def one(c, bv, bf):
            s_vec = 16 * (c % 16) + lanes
            col = jnp.broadcast_to(59 + c // 16, (16,))
            v = plsc.load_gather(conf_v, [base + s_vec, col])
            sig = 1.0 / (1.0 + jnp.exp(-v))
            f = 5 * s_vec + (c // 16)
            better = (sig > bv) | ((sig == bv) & (f < bf))
            return jnp.where(better, sig, bv), jnp.where(better, f, bf)

        def amax_body(j, carry):
            out = []
            for u in range(4):
                bv, bf = carry[2 * u], carry[2 * u + 1]
                out.extend(one(j + 20 * u, bv, bf))
            return tuple(out)

        init = []
        for _ in range(4):
            init.extend((jnp.full((16,), -1.0, jnp.float32),
                         jnp.zeros((16,), jnp.int32)))
        acc = lax.fori_loop(0, 20, amax_body, tuple(init))
        bv, bf = acc[0], acc[1]
        for u in range(1, 4):
            ov, of = acc[2 * u], acc[2 * u + 1]
            better = (ov > bv) | ((ov == bv) & (of < bf))
            bv = jnp.where(better, ov, bv)
            bf = jnp.where(better, of, bf)

        for k in (1, 2, 4, 8):
            red_f[...] = bv
            red_i[...] = bf
            perm = lanes ^ k
            ov = plsc.load_gather(red_f, [perm])
            of = plsc.load_gather(red_i, [perm])
            better = (ov > bv) | ((ov == bv) & (of < bf))
            bv = jnp.where(better, ov, bv)
            bf = jnp.where(better, of, bf)
        fm = bf[0]
        return fm % _DBINS, fm // _DBINS

    cc0.wait()
    d0, s0 = amax(0)
    cp0 = pltpu.make_async_copy(
        pred_hbm.at[pl.ds(row0 + (s0 // 8) * 8, 8), pl.ds(256, _NCELL)],
        cell0_v, semg0)
    cp0.start()
    cc1.wait()
    d1, s1 = amax(1)
    cp1 = pltpu.make_async_copy(
        pred_hbm.at[pl.ds(row1 + (s1 // 8) * 8, 8), pl.ds(256, _NCELL)],
        cell1_v, semg1)
    cp1.start()

    for i, (cp, cell_v, d, s) in enumerate(
            ((cp0, cell0_v, d0, s0), (cp1, cell1_v, d1, s1))):
        cp.wait()
        srow = jnp.broadcast_to(s % 8, (16,))

        def sel_body(k, carry):
            idxc = (_CLS0 - 256) + d + 5 * (16 * k + lanes)
            idxc = jnp.minimum(idxc, _NCELL - 1)
            rows_v[pl.ds(k * 16, 16)] = plsc.load_gather(cell_v, [srow, idxc])
            return carry

        lax.fori_loop(0, _NPAD // 16, sel_body, 0)
        pltpu.sync_copy(rows_v, out_hbm.at[b0 + i])


@functools.cache
def _sc_gather():
    return pl.kernel(
        _sc_gather_body,
        out_type=jax.ShapeDtypeStruct((_BS, _NPAD), jnp.float32),
        mesh=plsc.VectorSubcoreMesh(
            core_axis_name="c", subcore_axis_name="s",
            num_cores=2, num_subcores=16),
        compiler_params=pltpu.CompilerParams(
            needs_layout_passes=False, use_tc_tiling_on_sc=True),
        scratch_types=[
            pltpu.VMEM((2 * _SPATIAL, 128), jnp.float32),
            pltpu.VMEM((8, _NCELL), jnp.float32),
            pltpu.VMEM((8, _NCELL), jnp.float32),
            pltpu.VMEM((_NPAD,), jnp.float32),
            pltpu.VMEM((16,), jnp.float32),
            pltpu.VMEM((16,), jnp.int32),
            pltpu.SemaphoreType.DMA,
            pltpu.SemaphoreType.DMA,
            pltpu.SemaphoreType.DMA,
            pltpu.SemaphoreType.DMA,
        ],
    )


def _ce_body(chosen_ref, ga_ref, gn_ref, out_ref):
    x = chosen_ref[...]
    lane = lax.broadcasted_iota(jnp.int32, (_BS, _NPAD), 1)
    mask_a = lane < _NUM_ACTION
    mask_n = (lane >= _NUM_ACTION) & (lane < _NCLS)
    neg = jnp.float32(-1e30)
    xa = jnp.where(mask_a, x, neg)
    xn = jnp.where(mask_n, x, neg)
    ma = jnp.max(xa, axis=1, keepdims=True)
    mn = jnp.max(xn, axis=1, keepdims=True)
    sa = jnp.sum(jnp.where(mask_a, jnp.exp(xa - ma), 0.0), axis=1, keepdims=True)
    sn = jnp.sum(jnp.where(mask_n, jnp.exp(xn - mn), 0.0), axis=1, keepdims=True)
    lse_a = ma + jnp.log(sa)
    lse_n = mn + jnp.log(sn)
    pa = jnp.sum(jnp.where(lane == ga_ref[...], x, 0.0), axis=1, keepdims=True)
    pn = jnp.sum(jnp.where(lane == gn_ref[...] + _NUM_ACTION, x, 0.0),
                 axis=1, keepdims=True)
    la = jnp.sum(lse_a - pa) * 0.5
    ln = jnp.sum(lse_n - pn) * 0.5
    out_ref[0] = la + ln
    out_ref[1] = la
    out_ref[2] = ln


_ce_call = pl.pallas_call(
    _ce_body,
    out_shape=jax.ShapeDtypeStruct((3,), jnp.float32),
    out_specs=pl.BlockSpec(memory_space=pltpu.MemorySpace.SMEM),
)


def kernel(pred, action_gt, noun_gt):
    pred_m = jnp.transpose(pred, (0, 2, 3, 1)).reshape(_BS * _SPATIAL, _NCH)
    chosen = _sc_gather()(pred_m)
    ga = action_gt.astype(jnp.int32).reshape(_BS, 1)
    gn = noun_gt.astype(jnp.int32).reshape(_BS, 1)
    out = _ce_call(chosen, ga, gn)
    return (out[0], out[1], out[2])
